# baseline (speedup 1.0000x reference)
import functools

import jax
import jax.numpy as jnp
from jax import lax
from jax.experimental import pallas as pl
from jax.experimental.pallas import tpu as pltpu
from jax.experimental.pallas import tpu_sc as plsc

ROWS = 128
COLS = 32768
LANES = 16
GROUP = 4 * LANES
SC_ROWS = 32
TC_ROWS = ROWS - SC_ROWS
TC_BLOCK = 8
NEG_BIG = float("-inf")
POS_BIG = float("inf")


def _insert_max4(m, v):
    m1, m2, m3, m4 = m
    n1 = jnp.maximum(m1, v)
    t = jnp.minimum(m1, v)
    n2 = jnp.maximum(m2, t)
    t = jnp.minimum(m2, t)
    n3 = jnp.maximum(m3, t)
    t = jnp.minimum(m3, t)
    n4 = jnp.maximum(m4, t)
    return (n1, n2, n3, n4)


def _insert_min4(m, v):
    m1, m2, m3, m4 = m
    n1 = jnp.minimum(m1, v)
    t = jnp.maximum(m1, v)
    n2 = jnp.minimum(m2, t)
    t = jnp.maximum(m2, t)
    n3 = jnp.minimum(m3, t)
    t = jnp.maximum(m3, t)
    n4 = jnp.minimum(m4, t)
    return (n1, n2, n3, n4)


def _insert_max2(m, v):
    m1, m2 = m
    n1 = jnp.maximum(m1, v)
    t = jnp.minimum(m1, v)
    n2 = jnp.maximum(m2, t)
    return (n1, n2)


def _insert_min2(m, v):
    m1, m2 = m
    n1 = jnp.minimum(m1, v)
    t = jnp.maximum(m1, v)
    n2 = jnp.minimum(m2, t)
    return (n1, n2)


def _sort4(a, b, c, d):
    a1 = jnp.maximum(a, b)
    a2 = jnp.minimum(a, b)
    b1 = jnp.maximum(c, d)
    b2 = jnp.minimum(c, d)
    w1 = jnp.maximum(a1, b1)
    t1 = jnp.minimum(a1, b1)
    w4 = jnp.minimum(a2, b2)
    t2 = jnp.maximum(a2, b2)
    w2 = jnp.maximum(t1, t2)
    w3 = jnp.minimum(t1, t2)
    return w1, w2, w3, w4


def _tiered_step(carry, w1, w2, w3, w4):
    (m1, m2, m3, m4, p1, p2, q1, r1,
     u1, u2, u3, u4, s1, s2, e1, f1) = carry
    (m1, m2, m3, m4) = _insert_max4((m1, m2, m3, m4), w1)
    (p1, p2) = _insert_max2((p1, p2), w2)
    q1 = jnp.maximum(q1, w3)
    r1 = jnp.maximum(r1, w4)
    (u1, u2, u3, u4) = _insert_min4((u1, u2, u3, u4), w4)
    (s1, s2) = _insert_min2((s1, s2), w3)
    e1 = jnp.minimum(e1, w2)
    f1 = jnp.minimum(f1, w1)
    return (m1, m2, m3, m4, p1, p2, q1, r1,
            u1, u2, u3, u4, s1, s2, e1, f1)


def _make_sc_kernel():
    info = plsc.get_sparse_core_info()
    nc, ns = info.num_cores, info.num_subcores
    nw = nc * ns
    assert SC_ROWS <= nw
    n_chunks = 2
    chunk = COLS // n_chunks
    n_iters = chunk // GROUP
    mesh = plsc.VectorSubcoreMesh(core_axis_name="c", subcore_axis_name="s")

    @functools.partial(
        pl.kernel,
        mesh=mesh,
        out_type=jax.ShapeDtypeStruct((SC_ROWS, LANES), jnp.float32),
        scratch_types=[
            pltpu.VMEM((chunk,), jnp.float32),
            pltpu.VMEM((chunk,), jnp.float32),
            pltpu.VMEM((LANES,), jnp.float32),
            pltpu.SemaphoreType.DMA,
            pltpu.SemaphoreType.DMA,
        ],
        compiler_params=pltpu.CompilerParams(needs_layout_passes=False),
    )
    def topk_sc(x_hbm, out_hbm, buf0, buf1, outv, sem0, sem1):
        wid = lax.axis_index("s") * nc + lax.axis_index("c")
        iota = lax.iota(jnp.int32, LANES)
        bufs = (buf0, buf1)
        sems = (sem0, sem1)

        @pl.when(wid < SC_ROWS)
        def _():
            row = wid
            handle = pltpu.async_copy(
                x_hbm.at[row, pl.ds(0, chunk)], bufs[0], sems[0])
            neg = jnp.full((LANES,), NEG_BIG, jnp.float32)
            pos = jnp.full((LANES,), POS_BIG, jnp.float32)
            carry = (neg,) * 8 + (pos,) * 8
            for ch in range(n_chunks):
                cur = bufs[ch % 2]
                if ch + 1 < n_chunks:
                    nxt_handle = pltpu.async_copy(
                        x_hbm.at[row, pl.ds((ch + 1) * chunk, chunk)],
                        bufs[(ch + 1) % 2],
                        sems[(ch + 1) % 2],
                    )
                handle.wait()

                def body(i, carry, cur=cur):
                    base = i * GROUP
                    a = cur[pl.ds(base, LANES)]
                    b = cur[pl.ds(base + LANES, LANES)]
                    c = cur[pl.ds(base + 2 * LANES, LANES)]
                    d = cur[pl.ds(base + 3 * LANES, LANES)]
                    w1, w2, w3, w4 = _sort4(a, b, c, d)
                    return _tiered_step(carry, w1, w2, w3, w4)

                carry = lax.fori_loop(0, n_iters, body, carry, unroll=4)
                if ch + 1 < n_chunks:
                    handle = nxt_handle

            max_c = carry[0:8]
            min_c = carry[8:16]
            hi_s = [jnp.sort(v) for v in max_c]
            lo_s = [jnp.sort(v) for v in min_c]

            l1 = l2 = l3 = l4 = jnp.full((LANES,), NEG_BIG, jnp.float32)
            s1 = s2 = s3 = s4 = jnp.full((LANES,), POS_BIG, jnp.float32)
            for j in range(8):
                for t in range(4):
                    v = jnp.full((LANES,), hi_s[j][15 - t], jnp.float32)
                    (l1, l2, l3, l4) = _insert_max4((l1, l2, l3, l4), v)
                    w = jnp.full((LANES,), lo_s[j][t], jnp.float32)
                    (s1, s2, s3, s4) = _insert_min4((s1, s2, s3, s4), w)

            res = jnp.where(iota == 0, l1, jnp.float32(0.0))
            res = jnp.where(iota == 1, l2, res)
            res = jnp.where(iota == 2, l3, res)
            res = jnp.where(iota == 3, l4, res)
            res = jnp.where(iota == 4, s1, res)
            res = jnp.where(iota == 5, s2, res)
            res = jnp.where(iota == 6, s3, res)
            res = jnp.where(iota == 7, s4, res)
            outv[...] = res
            pltpu.sync_copy(outv, out_hbm.at[row])

    return topk_sc


def _merge_top4(a, b):
    return tuple(jnp.maximum(a[i], b[3 - i]) for i in range(4))


def _merge_bot4(a, b):
    return tuple(jnp.minimum(a[i], b[3 - i]) for i in range(4))


def _resort4_desc(w):
    a = jnp.maximum(w[0], w[2])
    c = jnp.minimum(w[0], w[2])
    b = jnp.maximum(w[1], w[3])
    d = jnp.minimum(w[1], w[3])
    return (jnp.maximum(a, b), jnp.minimum(a, b),
            jnp.maximum(c, d), jnp.minimum(c, d))


def _resort4_asc(w):
    a = jnp.minimum(w[0], w[2])
    c = jnp.maximum(w[0], w[2])
    b = jnp.minimum(w[1], w[3])
    d = jnp.maximum(w[1], w[3])
    return (jnp.minimum(a, b), jnp.maximum(a, b),
            jnp.minimum(c, d), jnp.maximum(c, d))


TC_CHUNK = 4096
TC_NCH = COLS // TC_CHUNK


def _tc_body(x_ref, o_ref, acc):
    j = pl.program_id(1)
    neg = jnp.full((TC_BLOCK, 128), NEG_BIG, jnp.float32)
    pos = jnp.full((TC_BLOCK, 128), POS_BIG, jnp.float32)

    def read_state():
        if_first = j == 0
        vals = []
        for k in range(32):
            v = acc[:, k * 128 : (k + 1) * 128]
            init = neg if (k % 16) < 8 else pos
            vals.append(jnp.where(if_first, init, v))
        return tuple(vals[0:16]), tuple(vals[16:32])

    sa, sb = read_state()
    n_groups = TC_CHUNK // (8 * 128)
    for g in range(n_groups):
        base = g * (8 * 128)
        va = [x_ref[:, base + s * 128 : base + (s + 1) * 128] for s in range(4)]
        vb = [x_ref[:, base + (s + 4) * 128 : base + (s + 5) * 128] for s in range(4)]
        sa = _tiered_step(sa, *_sort4(*va))
        sb = _tiered_step(sb, *_sort4(*vb))
    state = sa + sb
    for k in range(32):
        acc[:, k * 128 : (k + 1) * 128] = state[k]

    @pl.when(j == TC_NCH - 1)
    def _():
        def set_top(f):
            m = f[0:4]
            o = _sort4(f[4], f[5], f[6], f[7])
            return _resort4_desc(_merge_top4(m, o))

        def set_bot(f):
            u = f[8:12]
            o = _sort4(f[12], f[13], f[14], f[15])
            return _resort4_asc(_merge_bot4(u, o[::-1]))

        hi = _resort4_desc(_merge_top4(set_top(sa), set_top(sb)))
        lo = _resort4_asc(_merge_bot4(set_bot(sa), set_bot(sb)))

        for s in (64, 32, 16, 8, 4, 2, 1):
            rh = tuple(pltpu.roll(v, s, 1) for v in hi)
            hi = _resort4_desc(_merge_top4(hi, rh))
            rl = tuple(pltpu.roll(v, s, 1) for v in lo)
            lo = _resort4_asc(_merge_bot4(lo, rl))

        lane = lax.broadcasted_iota(jnp.int32, (TC_BLOCK, 128), 1)
        res = jnp.zeros((TC_BLOCK, 128), jnp.float32)
        for k in range(4):
            res = jnp.where(lane == k, hi[k], res)
            res = jnp.where(lane == 4 + k, lo[k], res)
        o_ref[...] = res


def _make_tc_kernel():
    grid = (TC_ROWS // TC_BLOCK, TC_NCH)
    return pl.pallas_call(
        _tc_body,
        grid=grid,
        in_specs=[
            pl.BlockSpec(
                (TC_BLOCK, TC_CHUNK),
                lambda i, j: (i + SC_ROWS // TC_BLOCK, j),
            )
        ],
        out_specs=pl.BlockSpec((TC_BLOCK, 128), lambda i, j: (i, 0)),
        out_shape=jax.ShapeDtypeStruct((TC_ROWS, 128), jnp.float32),
        scratch_shapes=[pltpu.VMEM((TC_BLOCK, 32 * 128), jnp.float32)],
        compiler_params=pltpu.CompilerParams(
            dimension_semantics=("parallel", "arbitrary")),
    )


_topk_sc = _make_sc_kernel()
_topk_tc = _make_tc_kernel()


@jax.jit
def kernel(x):
    sc_res = _topk_sc(x)
    tc_res = _topk_tc(x)
    largest = jnp.concatenate([sc_res[:, 0:4], tc_res[:, 0:4]], axis=0)
    smallest = jnp.concatenate([sc_res[:, 4:8], tc_res[:, 4:8]], axis=0)
    return (largest, smallest)


# speedup vs baseline: 2.1824x; 2.1824x over previous
import functools

import jax
import jax.numpy as jnp
from jax import lax
from jax.experimental import pallas as pl
from jax.experimental.pallas import tpu as pltpu
from jax.experimental.pallas import tpu_sc as plsc

ROWS = 128
COLS = 32768
LANES = 16
GROUP = 4 * LANES
SC_ROWS = 32
TC_ROWS = ROWS - SC_ROWS
TC_BLOCK = 8
NEG_BIG = float("-inf")
POS_BIG = float("inf")


def _insert_max4(m, v):
    m1, m2, m3, m4 = m
    n1 = jnp.maximum(m1, v)
    t = jnp.minimum(m1, v)
    n2 = jnp.maximum(m2, t)
    t = jnp.minimum(m2, t)
    n3 = jnp.maximum(m3, t)
    t = jnp.minimum(m3, t)
    n4 = jnp.maximum(m4, t)
    return (n1, n2, n3, n4)


def _insert_min4(m, v):
    m1, m2, m3, m4 = m
    n1 = jnp.minimum(m1, v)
    t = jnp.maximum(m1, v)
    n2 = jnp.minimum(m2, t)
    t = jnp.maximum(m2, t)
    n3 = jnp.minimum(m3, t)
    t = jnp.maximum(m3, t)
    n4 = jnp.minimum(m4, t)
    return (n1, n2, n3, n4)


def _insert_max2(m, v):
    m1, m2 = m
    n1 = jnp.maximum(m1, v)
    t = jnp.minimum(m1, v)
    n2 = jnp.maximum(m2, t)
    return (n1, n2)


def _insert_min2(m, v):
    m1, m2 = m
    n1 = jnp.minimum(m1, v)
    t = jnp.maximum(m1, v)
    n2 = jnp.minimum(m2, t)
    return (n1, n2)


def _sort4(a, b, c, d):
    a1 = jnp.maximum(a, b)
    a2 = jnp.minimum(a, b)
    b1 = jnp.maximum(c, d)
    b2 = jnp.minimum(c, d)
    w1 = jnp.maximum(a1, b1)
    t1 = jnp.minimum(a1, b1)
    w4 = jnp.minimum(a2, b2)
    t2 = jnp.maximum(a2, b2)
    w2 = jnp.maximum(t1, t2)
    w3 = jnp.minimum(t1, t2)
    return w1, w2, w3, w4


def _tiered_step(carry, w1, w2, w3, w4):
    (m1, m2, m3, m4, p1, p2, q1, r1,
     u1, u2, u3, u4, s1, s2, e1, f1) = carry
    (m1, m2, m3, m4) = _insert_max4((m1, m2, m3, m4), w1)
    (p1, p2) = _insert_max2((p1, p2), w2)
    q1 = jnp.maximum(q1, w3)
    r1 = jnp.maximum(r1, w4)
    (u1, u2, u3, u4) = _insert_min4((u1, u2, u3, u4), w4)
    (s1, s2) = _insert_min2((s1, s2), w3)
    e1 = jnp.minimum(e1, w2)
    f1 = jnp.minimum(f1, w1)
    return (m1, m2, m3, m4, p1, p2, q1, r1,
            u1, u2, u3, u4, s1, s2, e1, f1)


def _make_sc_kernel():
    info = plsc.get_sparse_core_info()
    nc, ns = info.num_cores, info.num_subcores
    nw = nc * ns
    assert SC_ROWS <= nw
    n_chunks = 2
    chunk = COLS // n_chunks
    n_iters = chunk // GROUP
    mesh = plsc.VectorSubcoreMesh(core_axis_name="c", subcore_axis_name="s")

    @functools.partial(
        pl.kernel,
        mesh=mesh,
        out_type=jax.ShapeDtypeStruct((SC_ROWS, LANES), jnp.float32),
        scratch_types=[
            pltpu.VMEM((chunk,), jnp.float32),
            pltpu.VMEM((chunk,), jnp.float32),
            pltpu.VMEM((LANES,), jnp.float32),
            pltpu.SemaphoreType.DMA,
            pltpu.SemaphoreType.DMA,
        ],
        compiler_params=pltpu.CompilerParams(needs_layout_passes=False),
    )
    def topk_sc(x_hbm, out_hbm, buf0, buf1, outv, sem0, sem1):
        wid = lax.axis_index("s") * nc + lax.axis_index("c")
        iota = lax.iota(jnp.int32, LANES)
        bufs = (buf0, buf1)
        sems = (sem0, sem1)

        @pl.when(wid < SC_ROWS)
        def _():
            row = wid
            handle = pltpu.async_copy(
                x_hbm.at[row, pl.ds(0, chunk)], bufs[0], sems[0])
            neg = jnp.full((LANES,), NEG_BIG, jnp.float32)
            pos = jnp.full((LANES,), POS_BIG, jnp.float32)
            carry = (neg,) * 8 + (pos,) * 8
            for ch in range(n_chunks):
                cur = bufs[ch % 2]
                if ch + 1 < n_chunks:
                    nxt_handle = pltpu.async_copy(
                        x_hbm.at[row, pl.ds((ch + 1) * chunk, chunk)],
                        bufs[(ch + 1) % 2],
                        sems[(ch + 1) % 2],
                    )
                handle.wait()

                def body(i, carry, cur=cur):
                    base = i * GROUP
                    a = cur[pl.ds(base, LANES)]
                    b = cur[pl.ds(base + LANES, LANES)]
                    c = cur[pl.ds(base + 2 * LANES, LANES)]
                    d = cur[pl.ds(base + 3 * LANES, LANES)]
                    w1, w2, w3, w4 = _sort4(a, b, c, d)
                    return _tiered_step(carry, w1, w2, w3, w4)

                carry = lax.fori_loop(0, n_iters, body, carry, unroll=4)
                if ch + 1 < n_chunks:
                    handle = nxt_handle

            max_c = carry[0:8]
            min_c = carry[8:16]
            hi_s = [jnp.sort(v) for v in max_c]
            lo_s = [jnp.sort(v) for v in min_c]

            l1 = l2 = l3 = l4 = jnp.full((LANES,), NEG_BIG, jnp.float32)
            s1 = s2 = s3 = s4 = jnp.full((LANES,), POS_BIG, jnp.float32)
            for j in range(8):
                for t in range(4):
                    v = jnp.full((LANES,), hi_s[j][15 - t], jnp.float32)
                    (l1, l2, l3, l4) = _insert_max4((l1, l2, l3, l4), v)
                    w = jnp.full((LANES,), lo_s[j][t], jnp.float32)
                    (s1, s2, s3, s4) = _insert_min4((s1, s2, s3, s4), w)

            res = jnp.where(iota == 0, l1, jnp.float32(0.0))
            res = jnp.where(iota == 1, l2, res)
            res = jnp.where(iota == 2, l3, res)
            res = jnp.where(iota == 3, l4, res)
            res = jnp.where(iota == 4, s1, res)
            res = jnp.where(iota == 5, s2, res)
            res = jnp.where(iota == 6, s3, res)
            res = jnp.where(iota == 7, s4, res)
            outv[...] = res
            pltpu.sync_copy(outv, out_hbm.at[row])

    return topk_sc


def _merge_top4(a, b):
    return tuple(jnp.maximum(a[i], b[3 - i]) for i in range(4))


def _merge_bot4(a, b):
    return tuple(jnp.minimum(a[i], b[3 - i]) for i in range(4))


def _resort4_desc(w):
    a = jnp.maximum(w[0], w[2])
    c = jnp.minimum(w[0], w[2])
    b = jnp.maximum(w[1], w[3])
    d = jnp.minimum(w[1], w[3])
    return (jnp.maximum(a, b), jnp.minimum(a, b),
            jnp.maximum(c, d), jnp.minimum(c, d))


def _resort4_asc(w):
    a = jnp.minimum(w[0], w[2])
    c = jnp.maximum(w[0], w[2])
    b = jnp.minimum(w[1], w[3])
    d = jnp.maximum(w[1], w[3])
    return (jnp.minimum(a, b), jnp.maximum(a, b),
            jnp.minimum(c, d), jnp.maximum(c, d))


def _tc_body(x_ref, o_ref):
    neg = jnp.full((TC_BLOCK, 128), NEG_BIG, jnp.float32)
    pos = jnp.full((TC_BLOCK, 128), POS_BIG, jnp.float32)
    init = (neg,) * 8 + (pos,) * 8
    sa, sb = init, init
    for g in range(COLS // (8 * 128)):
        base = g * (8 * 128)
        va = [x_ref[:, base + s * 128 : base + (s + 1) * 128] for s in range(4)]
        vb = [x_ref[:, base + (s + 4) * 128 : base + (s + 5) * 128] for s in range(4)]
        sa = _tiered_step(sa, *_sort4(*va))
        sb = _tiered_step(sb, *_sort4(*vb))

    def set_top(f):
        m = f[0:4]
        o = _sort4(f[4], f[5], f[6], f[7])
        return _resort4_desc(_merge_top4(m, o))

    def set_bot(f):
        u = f[8:12]
        o = _sort4(f[12], f[13], f[14], f[15])
        return _resort4_asc(_merge_bot4(u, o[::-1]))

    hi = _resort4_desc(_merge_top4(set_top(sa), set_top(sb)))
    lo = _resort4_asc(_merge_bot4(set_bot(sa), set_bot(sb)))

    for s in (64, 32, 16, 8, 4, 2, 1):
        rh = tuple(pltpu.roll(v, s, 1) for v in hi)
        hi = _resort4_desc(_merge_top4(hi, rh))
        rl = tuple(pltpu.roll(v, s, 1) for v in lo)
        lo = _resort4_asc(_merge_bot4(lo, rl))

    lane = lax.broadcasted_iota(jnp.int32, (TC_BLOCK, 128), 1)
    res = jnp.zeros((TC_BLOCK, 128), jnp.float32)
    for k in range(4):
        res = jnp.where(lane == k, hi[k], res)
        res = jnp.where(lane == 4 + k, lo[k], res)
    o_ref[...] = res


def _make_tc_kernel():
    grid = (TC_ROWS // TC_BLOCK,)
    return pl.pallas_call(
        _tc_body,
        grid=grid,
        in_specs=[
            pl.BlockSpec(
                (TC_BLOCK, COLS),
                lambda i: (i + SC_ROWS // TC_BLOCK, 0),
            )
        ],
        out_specs=pl.BlockSpec((TC_BLOCK, 128), lambda i: (i, 0)),
        out_shape=jax.ShapeDtypeStruct((TC_ROWS, 128), jnp.float32),
        compiler_params=pltpu.CompilerParams(
            dimension_semantics=("arbitrary",)),
    )


_topk_sc = _make_sc_kernel()
_topk_tc = _make_tc_kernel()


@jax.jit
def kernel(x):
    sc_res = _topk_sc(x)
    tc_res = _topk_tc(x)
    largest = jnp.concatenate([sc_res[:, 0:4], tc_res[:, 0:4]], axis=0)
    smallest = jnp.concatenate([sc_res[:, 4:8], tc_res[:, 4:8]], axis=0)
    return (largest, smallest)
